# baseline (speedup 1.0000x reference)
import jax
import jax.numpy as jnp
from jax import lax
from jax.experimental import pallas as pl
from jax.experimental.pallas import tpu as pltpu

_BK = 256


def _fmt_anch_body(in_ref, out_ref):
    at = in_ref[...].T.reshape(8, 4, in_ref.shape[0])
    for f in range(4):
        out_ref[f, :, :] = at[:, f, :]


def _fmt_reg_body(in_ref, out_ref):
    at = in_ref[0].T.reshape(8, 4, in_ref.shape[1])
    for f in range(4):
        out_ref[0, f, :, :] = at[:, f, :]


def _focal_body(cls_ref, reg_ref, anch_ref, ann_ref, nvalid_ref, out_ref):
    i = pl.program_id(1)
    j = pl.program_id(0)

    xt = cls_ref[0].T
    C = 80
    K = xt.shape[1]
    x = jnp.clip(xt, 1e-4, 1.0 - 1e-4).reshape(8, C, K)

    ax1 = anch_ref[0]
    ay1 = anch_ref[1]
    ax2 = anch_ref[2]
    ay2 = anch_ref[3]
    aw = ax2 - ax1
    ah = ay2 - ay1
    acx = ax1 + 0.5 * aw
    acy = ay1 + 0.5 * ah
    aw_s = jnp.maximum(aw, 1e-3)
    ah_s = jnp.maximum(ah, 1e-3)
    area_a = aw * ah

    M = ann_ref.shape[1]
    neg_one = jnp.float32(-1.0)
    best = jnp.full((8, K), neg_one)
    gx1 = jnp.zeros((8, K), jnp.float32)
    gy1 = jnp.zeros((8, K), jnp.float32)
    gx2 = jnp.zeros((8, K), jnp.float32)
    gy2 = jnp.zeros((8, K), jnp.float32)
    glab = jnp.zeros((8, K), jnp.float32)
    for m in range(M):
        bx1 = ann_ref[0, m, 0]
        by1 = ann_ref[0, m, 1]
        bx2 = ann_ref[0, m, 2]
        by2 = ann_ref[0, m, 3]
        blab = ann_ref[0, m, 4]
        iw = jnp.maximum(jnp.minimum(ax2, bx2) - jnp.maximum(ax1, bx1), 0.0)
        ih = jnp.maximum(jnp.minimum(ay2, by2) - jnp.maximum(ay1, by1), 0.0)
        inter = iw * ih
        area_b = (bx2 - bx1) * (by2 - by1)
        ua = jnp.maximum(area_a + area_b - inter, 1e-8)
        iou = inter / ua
        cond = iou > best
        best = jnp.where(cond, iou, best)
        gx1 = jnp.where(cond, bx1, gx1)
        gy1 = jnp.where(cond, by1, gy1)
        gx2 = jnp.where(cond, bx2, gx2)
        gy2 = jnp.where(cond, by2, gy2)
        glab = jnp.where(cond, blab, glab)

    nvalid = nvalid_ref[0]
    sub_i = lax.broadcasted_iota(jnp.int32, (8, K), 0)
    lane_i = lax.broadcasted_iota(jnp.int32, (8, K), 1)
    valid = (i * (8 * K) + 8 * lane_i + sub_i) < nvalid
    pos = (best >= 0.5) & valid
    contrib = ((best >= 0.5) | (best < 0.4)) & valid
    posf = pos.astype(jnp.float32)
    npos = jnp.sum(posf)

    neg = (0.75 * (x * x)) * (-jnp.log(1.0 - x))
    s_neg = jnp.sum(neg, axis=1)
    lab_i = glab.astype(jnp.int32).reshape(8, 1, K)
    iota_c = lax.broadcasted_iota(jnp.int32, (8, C, K), 1)
    x_sel = jnp.sum(jnp.where(iota_c == lab_i, x, 0.0), axis=1)
    pos_term = (0.25 * (1.0 - x_sel) * (1.0 - x_sel)) * (-jnp.log(x_sel))
    neg_sel = (0.75 * (x_sel * x_sel)) * (-jnp.log(1.0 - x_sel))
    row_cls = (jnp.where(contrib, s_neg, 0.0)
               + jnp.where(pos, pos_term - neg_sel, 0.0))
    cls_s = jnp.sum(row_cls)

    gt_w = gx2 - gx1
    gt_h = gy2 - gy1
    gcx = gx1 + 0.5 * gt_w
    gcy = gy1 + 0.5 * gt_h
    gt_w = jnp.maximum(gt_w, 1.0)
    gt_h = jnp.maximum(gt_h, 1.0)
    tdx = ((gcx - acx) / aw_s) / 0.1
    tdy = ((gcy - acy) / ah_s) / 0.1
    tdw = jnp.log(gt_w / aw_s) / 0.2
    tdh = jnp.log(gt_h / ah_s) / 0.2

    def smooth_l1(t, c):
        d = jnp.abs(t - reg_ref[0, c])
        return jnp.where(d <= 1.0 / 9.0, 0.5 * 9.0 * (d * d), d - 0.5 / 9.0)

    rl = smooth_l1(tdx, 0) + smooth_l1(tdy, 1) + smooth_l1(tdw, 2) + smooth_l1(tdh, 3)
    reg_s = jnp.sum(jnp.where(pos, rl, 0.0))

    @pl.when(i == 0)
    def _init():
        out_ref[0, 0, 0] = 0.0
        out_ref[0, 0, 1] = 0.0
        out_ref[0, 0, 2] = 0.0
        out_ref[0, 0, 3] = 0.0

    out_ref[0, 0, 0] += cls_s
    out_ref[0, 0, 1] += reg_s
    out_ref[0, 0, 2] += npos


@jax.jit
def kernel(classifications, regressions, anchors, annotations):
    B, A, C = classifications.shape
    M = annotations.shape[1]
    A8 = A // 8
    nblk = -(-A8 // _BK)

    cls_flat = classifications.reshape(B, A8, 8 * C)
    anch_r = pl.pallas_call(
        _fmt_anch_body,
        out_shape=jax.ShapeDtypeStruct((4, 8, A8), jnp.float32),
    )(anchors[0].reshape(A8, 32))
    reg_r = pl.pallas_call(
        _fmt_reg_body,
        grid=(B,),
        in_specs=[pl.BlockSpec((1, A8, 32), lambda j: (j, 0, 0))],
        out_specs=pl.BlockSpec((1, 4, 8, A8), lambda j: (j, 0, 0, 0)),
        out_shape=jax.ShapeDtypeStruct((B, 4, 8, A8), jnp.float32),
    )(regressions.reshape(B, A8, 32))
    nvalid = jnp.full((1,), A, dtype=jnp.int32)

    out = pl.pallas_call(
        _focal_body,
        grid=(B, nblk),
        in_specs=[
            pl.BlockSpec((1, _BK, 8 * C), lambda j, i: (j, i, 0)),
            pl.BlockSpec((1, 4, 8, _BK), lambda j, i: (j, 0, 0, i)),
            pl.BlockSpec((4, 8, _BK), lambda j, i: (0, 0, i)),
            pl.BlockSpec((1, M, 5), lambda j, i: (j, 0, 0),
                         memory_space=pltpu.SMEM),
            pl.BlockSpec(memory_space=pltpu.SMEM),
        ],
        out_specs=pl.BlockSpec((1, 1, 4), lambda j, i: (j, 0, 0),
                               memory_space=pltpu.SMEM),
        out_shape=jax.ShapeDtypeStruct((B, 1, 4), jnp.float32),
    )(cls_flat, reg_r, anch_r, annotations, nvalid)

    cls_sum = out[:, 0, 0]
    reg_sum = out[:, 0, 1]
    npos = out[:, 0, 2]
    cls_loss = jnp.mean(cls_sum / jnp.maximum(npos, 1.0)).reshape(1)
    reg_loss = jnp.mean(reg_sum / jnp.maximum(npos * 4.0, 1.0)).reshape(1)
    return cls_loss, reg_loss


# speedup vs baseline: 3.9007x; 3.9007x over previous
import jax
import jax.numpy as jnp
from jax import lax
from jax.experimental import pallas as pl
from jax.experimental.pallas import tpu as pltpu

_BA = 2048


def _focal_body(cls_ref, reg_ref, anch_ref, ann_ref, nvalid_ref, out_ref):
    i = pl.program_id(1)

    x = jnp.clip(cls_ref[0], 1e-4, 1.0 - 1e-4)
    C, BA = x.shape
    nvalid = nvalid_ref[0]
    valid = (lax.broadcasted_iota(jnp.int32, (1, BA), 1) + i * BA) < nvalid
    x = jnp.where(valid, x, 0.5)
    annb = ann_ref[0]
    M = annb.shape[0]
    bx1 = annb[:, 0:1]
    by1 = annb[:, 1:2]
    bx2 = annb[:, 2:3]
    by2 = annb[:, 3:4]

    ax1 = jnp.where(valid, anch_ref[0:1, :], 0.0)
    ay1 = jnp.where(valid, anch_ref[1:2, :], 0.0)
    ax2 = jnp.where(valid, anch_ref[2:3, :], 16.0)
    ay2 = jnp.where(valid, anch_ref[3:4, :], 16.0)
    aw = ax2 - ax1
    ah = ay2 - ay1
    acx = ax1 + 0.5 * aw
    acy = ay1 + 0.5 * ah
    aw_s = jnp.maximum(aw, 1e-3)
    ah_s = jnp.maximum(ah, 1e-3)

    area_a = aw * ah
    area_b = (bx2 - bx1) * (by2 - by1)
    iw = jnp.maximum(jnp.minimum(ax2, bx2) - jnp.maximum(ax1, bx1), 0.0)
    ih = jnp.maximum(jnp.minimum(ay2, by2) - jnp.maximum(ay1, by1), 0.0)
    inter = iw * ih
    ua = jnp.maximum(area_a + area_b - inter, 1e-8)
    iou = inter / ua

    iou_max = jnp.max(iou, axis=0, keepdims=True)
    iota_m = lax.broadcasted_iota(jnp.int32, (M, BA), 0)
    amax = jnp.min(jnp.where(iou == iou_max, iota_m, M), axis=0, keepdims=True)
    oh_m = iota_m == amax

    def pick(col):
        return jnp.sum(jnp.where(oh_m, col, 0.0), axis=0, keepdims=True)

    gx1 = pick(bx1)
    gy1 = pick(by1)
    gx2 = pick(bx2)
    gy2 = pick(by2)

    pos = (iou_max >= 0.5) & valid
    contrib = ((iou_max >= 0.5) | (iou_max < 0.4)) & valid
    posf = pos.astype(jnp.float32)
    npos = jnp.sum(posf)

    neg = (0.75 * (x * x)) * (-jnp.log(1.0 - x))
    s_neg = jax.lax.dot_general(jnp.ones((1, C), jnp.float32), neg,
                                (((1,), (0,)), ((), ())),
                                preferred_element_type=jnp.float32)
    blab = annb[:, 4:5]
    glab = pick(blab)
    lab_i = glab.astype(jnp.int32)
    iota_c = lax.broadcasted_iota(jnp.int32, (C, BA), 0)
    x_sel = jnp.sum(jnp.where(iota_c == lab_i, x, 0.0), axis=0, keepdims=True)
    pos_term = (0.25 * (1.0 - x_sel) * (1.0 - x_sel)) * (-jnp.log(x_sel))
    neg_sel = (0.75 * (x_sel * x_sel)) * (-jnp.log(1.0 - x_sel))
    row_cls = (jnp.where(contrib, s_neg, 0.0)
               + jnp.where(pos, pos_term - neg_sel, 0.0))
    cls_s = jnp.sum(row_cls)

    gt_w = gx2 - gx1
    gt_h = gy2 - gy1
    gcx = gx1 + 0.5 * gt_w
    gcy = gy1 + 0.5 * gt_h
    gt_w = jnp.maximum(gt_w, 1.0)
    gt_h = jnp.maximum(gt_h, 1.0)
    tdx = ((gcx - acx) / aw_s) / 0.1
    tdy = ((gcy - acy) / ah_s) / 0.1
    tdw = jnp.log(gt_w / aw_s) / 0.2
    tdh = jnp.log(gt_h / ah_s) / 0.2

    def smooth_l1(t, c):
        d = jnp.abs(t - reg_ref[0, c:c + 1, :])
        return jnp.where(d <= 1.0 / 9.0, 0.5 * 9.0 * (d * d), d - 0.5 / 9.0)

    rl = smooth_l1(tdx, 0) + smooth_l1(tdy, 1) + smooth_l1(tdw, 2) + smooth_l1(tdh, 3)
    reg_s = jnp.sum(jnp.where(pos, rl, 0.0))

    @pl.when(i == 0)
    def _init():
        out_ref[0, 0, 0] = 0.0
        out_ref[0, 0, 1] = 0.0
        out_ref[0, 0, 2] = 0.0
        out_ref[0, 0, 3] = 0.0

    out_ref[0, 0, 0] += cls_s
    out_ref[0, 0, 1] += reg_s
    out_ref[0, 0, 2] += npos


@jax.jit
def kernel(classifications, regressions, anchors, annotations):
    B, A, C = classifications.shape
    M = annotations.shape[1]
    nblk = -(-A // _BA)

    cls_t = classifications.transpose(0, 2, 1)
    reg_t = regressions.transpose(0, 2, 1)
    anch_t = anchors[0].T
    nvalid = jnp.full((1,), A, dtype=jnp.int32)

    out = pl.pallas_call(
        _focal_body,
        grid=(B, nblk),
        in_specs=[
            pl.BlockSpec((1, C, _BA), lambda j, i: (j, 0, i)),
            pl.BlockSpec((1, 4, _BA), lambda j, i: (j, 0, i)),
            pl.BlockSpec((4, _BA), lambda j, i: (0, i)),
            pl.BlockSpec((1, M, 5), lambda j, i: (j, 0, 0)),
            pl.BlockSpec(memory_space=pltpu.SMEM),
        ],
        out_specs=pl.BlockSpec((1, 1, 4), lambda j, i: (j, 0, 0),
                               memory_space=pltpu.SMEM),
        out_shape=jax.ShapeDtypeStruct((B, 1, 4), jnp.float32),
    )(cls_t, reg_t, anch_t, annotations, nvalid)

    cls_sum = out[:, 0, 0]
    reg_sum = out[:, 0, 1]
    npos = out[:, 0, 2]
    cls_loss = jnp.mean(cls_sum / jnp.maximum(npos, 1.0)).reshape(1)
    reg_loss = jnp.mean(reg_sum / jnp.maximum(npos * 4.0, 1.0)).reshape(1)
    return cls_loss, reg_loss


# speedup vs baseline: 4.2721x; 1.0952x over previous
import jax
import jax.numpy as jnp
from jax import lax
from jax.experimental import pallas as pl
from jax.experimental.pallas import tpu as pltpu

_BA = 4096


def _focal_body(cls_ref, reg_ref, anch_ref, ann_ref, nvalid_ref, out_ref):
    i = pl.program_id(1)

    x = jnp.clip(cls_ref[0], 1e-4, 1.0 - 1e-4)
    C, BA = x.shape
    nvalid = nvalid_ref[0]
    valid = (lax.broadcasted_iota(jnp.int32, (1, BA), 1) + i * BA) < nvalid
    x = jnp.where(valid, x, 0.5)
    annb = ann_ref[0]
    M = annb.shape[0]
    bx1 = annb[:, 0:1]
    by1 = annb[:, 1:2]
    bx2 = annb[:, 2:3]
    by2 = annb[:, 3:4]

    ax1 = jnp.where(valid, anch_ref[0:1, :], 0.0)
    ay1 = jnp.where(valid, anch_ref[1:2, :], 0.0)
    ax2 = jnp.where(valid, anch_ref[2:3, :], 16.0)
    ay2 = jnp.where(valid, anch_ref[3:4, :], 16.0)
    aw = ax2 - ax1
    ah = ay2 - ay1
    acx = ax1 + 0.5 * aw
    acy = ay1 + 0.5 * ah
    aw_s = jnp.maximum(aw, 1e-3)
    ah_s = jnp.maximum(ah, 1e-3)

    area_a = aw * ah
    area_b = (bx2 - bx1) * (by2 - by1)
    iw = jnp.maximum(jnp.minimum(ax2, bx2) - jnp.maximum(ax1, bx1), 0.0)
    ih = jnp.maximum(jnp.minimum(ay2, by2) - jnp.maximum(ay1, by1), 0.0)
    inter = iw * ih
    ua = jnp.maximum(area_a + area_b - inter, 1e-8)
    iou = inter / ua

    iou_max = jnp.max(iou, axis=0, keepdims=True)
    iota_m = lax.broadcasted_iota(jnp.int32, (M, BA), 0)
    amax = jnp.min(jnp.where(iou == iou_max, iota_m, M), axis=0, keepdims=True)
    oh_m = iota_m == amax

    def pick(col):
        return jnp.sum(jnp.where(oh_m, col, 0.0), axis=0, keepdims=True)

    gx1 = pick(bx1)
    gy1 = pick(by1)
    gx2 = pick(bx2)
    gy2 = pick(by2)

    pos = (iou_max >= 0.5) & valid
    contrib = ((iou_max >= 0.5) | (iou_max < 0.4)) & valid
    posf = pos.astype(jnp.float32)
    npos = jnp.sum(posf)

    neg = (0.75 * (x * x)) * (-jnp.log(1.0 - x))
    s_neg = jax.lax.dot_general(jnp.ones((1, C), jnp.float32), neg,
                                (((1,), (0,)), ((), ())),
                                preferred_element_type=jnp.float32)
    blab = annb[:, 4:5]
    glab = pick(blab)
    lab_i = glab.astype(jnp.int32)
    iota_c = lax.broadcasted_iota(jnp.int32, (C, BA), 0)
    x_sel = jnp.sum(jnp.where(iota_c == lab_i, x, 0.0), axis=0, keepdims=True)
    pos_term = (0.25 * (1.0 - x_sel) * (1.0 - x_sel)) * (-jnp.log(x_sel))
    neg_sel = (0.75 * (x_sel * x_sel)) * (-jnp.log(1.0 - x_sel))
    row_cls = (jnp.where(contrib, s_neg, 0.0)
               + jnp.where(pos, pos_term - neg_sel, 0.0))
    cls_s = jnp.sum(row_cls)

    gt_w = gx2 - gx1
    gt_h = gy2 - gy1
    gcx = gx1 + 0.5 * gt_w
    gcy = gy1 + 0.5 * gt_h
    gt_w = jnp.maximum(gt_w, 1.0)
    gt_h = jnp.maximum(gt_h, 1.0)
    tdx = ((gcx - acx) / aw_s) / 0.1
    tdy = ((gcy - acy) / ah_s) / 0.1
    tdw = jnp.log(gt_w / aw_s) / 0.2
    tdh = jnp.log(gt_h / ah_s) / 0.2

    def smooth_l1(t, c):
        d = jnp.abs(t - reg_ref[0, c:c + 1, :])
        return jnp.where(d <= 1.0 / 9.0, 0.5 * 9.0 * (d * d), d - 0.5 / 9.0)

    rl = smooth_l1(tdx, 0) + smooth_l1(tdy, 1) + smooth_l1(tdw, 2) + smooth_l1(tdh, 3)
    reg_s = jnp.sum(jnp.where(pos, rl, 0.0))

    @pl.when(i == 0)
    def _init():
        out_ref[0, 0, 0] = 0.0
        out_ref[0, 0, 1] = 0.0
        out_ref[0, 0, 2] = 0.0
        out_ref[0, 0, 3] = 0.0

    out_ref[0, 0, 0] += cls_s
    out_ref[0, 0, 1] += reg_s
    out_ref[0, 0, 2] += npos


@jax.jit
def kernel(classifications, regressions, anchors, annotations):
    B, A, C = classifications.shape
    M = annotations.shape[1]
    nblk = -(-A // _BA)

    cls_t = classifications.transpose(0, 2, 1)
    reg_t = regressions.transpose(0, 2, 1)
    anch_t = anchors[0].T
    nvalid = jnp.full((1,), A, dtype=jnp.int32)

    out = pl.pallas_call(
        _focal_body,
        grid=(B, nblk),
        in_specs=[
            pl.BlockSpec((1, C, _BA), lambda j, i: (j, 0, i)),
            pl.BlockSpec((1, 4, _BA), lambda j, i: (j, 0, i)),
            pl.BlockSpec((4, _BA), lambda j, i: (0, i)),
            pl.BlockSpec((1, M, 5), lambda j, i: (j, 0, 0)),
            pl.BlockSpec(memory_space=pltpu.SMEM),
        ],
        out_specs=pl.BlockSpec((1, 1, 4), lambda j, i: (j, 0, 0),
                               memory_space=pltpu.SMEM),
        out_shape=jax.ShapeDtypeStruct((B, 1, 4), jnp.float32),
    )(cls_t, reg_t, anch_t, annotations, nvalid)

    cls_sum = out[:, 0, 0]
    reg_sum = out[:, 0, 1]
    npos = out[:, 0, 2]
    cls_loss = jnp.mean(cls_sum / jnp.maximum(npos, 1.0)).reshape(1)
    reg_loss = jnp.mean(reg_sum / jnp.maximum(npos * 4.0, 1.0)).reshape(1)
    return cls_loss, reg_loss


# speedup vs baseline: 4.3671x; 1.0222x over previous
import jax
import jax.numpy as jnp
from jax import lax
from jax.experimental import pallas as pl
from jax.experimental.pallas import tpu as pltpu

_BA = 5120


def _focal_body(cls_ref, reg_ref, anch_ref, ann_ref, nvalid_ref, out_ref):
    i = pl.program_id(1)

    x = jnp.clip(cls_ref[0], 1e-4, 1.0 - 1e-4)
    C, BA = x.shape
    nvalid = nvalid_ref[0]
    valid = (lax.broadcasted_iota(jnp.int32, (1, BA), 1) + i * BA) < nvalid
    x = jnp.where(valid, x, 0.5)
    annb = ann_ref[0]
    M = annb.shape[0]
    bx1 = annb[:, 0:1]
    by1 = annb[:, 1:2]
    bx2 = annb[:, 2:3]
    by2 = annb[:, 3:4]

    ax1 = jnp.where(valid, anch_ref[0:1, :], 0.0)
    ay1 = jnp.where(valid, anch_ref[1:2, :], 0.0)
    ax2 = jnp.where(valid, anch_ref[2:3, :], 16.0)
    ay2 = jnp.where(valid, anch_ref[3:4, :], 16.0)
    aw = ax2 - ax1
    ah = ay2 - ay1
    acx = ax1 + 0.5 * aw
    acy = ay1 + 0.5 * ah
    aw_s = jnp.maximum(aw, 1e-3)
    ah_s = jnp.maximum(ah, 1e-3)

    area_a = aw * ah
    area_b = (bx2 - bx1) * (by2 - by1)
    iw = jnp.maximum(jnp.minimum(ax2, bx2) - jnp.maximum(ax1, bx1), 0.0)
    ih = jnp.maximum(jnp.minimum(ay2, by2) - jnp.maximum(ay1, by1), 0.0)
    inter = iw * ih
    ua = jnp.maximum(area_a + area_b - inter, 1e-8)
    iou = inter / ua

    iou_max = jnp.max(iou, axis=0, keepdims=True)
    iota_m = lax.broadcasted_iota(jnp.int32, (M, BA), 0)
    amax = jnp.min(jnp.where(iou == iou_max, iota_m, M), axis=0, keepdims=True)
    oh_m = iota_m == amax

    def pick(col):
        return jnp.sum(jnp.where(oh_m, col, 0.0), axis=0, keepdims=True)

    gx1 = pick(bx1)
    gy1 = pick(by1)
    gx2 = pick(bx2)
    gy2 = pick(by2)

    pos = (iou_max >= 0.5) & valid
    contrib = ((iou_max >= 0.5) | (iou_max < 0.4)) & valid
    posf = pos.astype(jnp.float32)
    npos = jnp.sum(posf)

    neg = (0.75 * (x * x)) * (-jnp.log(1.0 - x))
    s_neg = jax.lax.dot_general(jnp.ones((1, C), jnp.float32), neg,
                                (((1,), (0,)), ((), ())),
                                preferred_element_type=jnp.float32)
    blab = annb[:, 4:5]
    glab = pick(blab)
    lab_i = glab.astype(jnp.int32)
    iota_c = lax.broadcasted_iota(jnp.int32, (C, BA), 0)
    x_sel = jnp.sum(jnp.where(iota_c == lab_i, x, 0.0), axis=0, keepdims=True)
    pos_term = (0.25 * (1.0 - x_sel) * (1.0 - x_sel)) * (-jnp.log(x_sel))
    neg_sel = (0.75 * (x_sel * x_sel)) * (-jnp.log(1.0 - x_sel))
    row_cls = (jnp.where(contrib, s_neg, 0.0)
               + jnp.where(pos, pos_term - neg_sel, 0.0))
    cls_s = jnp.sum(row_cls)

    gt_w = gx2 - gx1
    gt_h = gy2 - gy1
    gcx = gx1 + 0.5 * gt_w
    gcy = gy1 + 0.5 * gt_h
    gt_w = jnp.maximum(gt_w, 1.0)
    gt_h = jnp.maximum(gt_h, 1.0)
    tdx = ((gcx - acx) / aw_s) / 0.1
    tdy = ((gcy - acy) / ah_s) / 0.1
    tdw = jnp.log(gt_w / aw_s) / 0.2
    tdh = jnp.log(gt_h / ah_s) / 0.2

    def smooth_l1(t, c):
        d = jnp.abs(t - reg_ref[0, c:c + 1, :])
        return jnp.where(d <= 1.0 / 9.0, 0.5 * 9.0 * (d * d), d - 0.5 / 9.0)

    rl = smooth_l1(tdx, 0) + smooth_l1(tdy, 1) + smooth_l1(tdw, 2) + smooth_l1(tdh, 3)
    reg_s = jnp.sum(jnp.where(pos, rl, 0.0))

    @pl.when(i == 0)
    def _init():
        out_ref[0, 0, 0] = 0.0
        out_ref[0, 0, 1] = 0.0
        out_ref[0, 0, 2] = 0.0
        out_ref[0, 0, 3] = 0.0

    out_ref[0, 0, 0] += cls_s
    out_ref[0, 0, 1] += reg_s
    out_ref[0, 0, 2] += npos


@jax.jit
def kernel(classifications, regressions, anchors, annotations):
    B, A, C = classifications.shape
    M = annotations.shape[1]
    nblk = -(-A // _BA)

    cls_t = classifications.transpose(0, 2, 1)
    reg_t = regressions.transpose(0, 2, 1)
    anch_t = anchors[0].T
    nvalid = jnp.full((1,), A, dtype=jnp.int32)

    out = pl.pallas_call(
        _focal_body,
        grid=(B, nblk),
        in_specs=[
            pl.BlockSpec((1, C, _BA), lambda j, i: (j, 0, i)),
            pl.BlockSpec((1, 4, _BA), lambda j, i: (j, 0, i)),
            pl.BlockSpec((4, _BA), lambda j, i: (0, i)),
            pl.BlockSpec((1, M, 5), lambda j, i: (j, 0, 0)),
            pl.BlockSpec(memory_space=pltpu.SMEM),
        ],
        out_specs=pl.BlockSpec((1, 1, 4), lambda j, i: (j, 0, 0),
                               memory_space=pltpu.SMEM),
        out_shape=jax.ShapeDtypeStruct((B, 1, 4), jnp.float32),
    )(cls_t, reg_t, anch_t, annotations, nvalid)

    cls_sum = out[:, 0, 0]
    reg_sum = out[:, 0, 1]
    npos = out[:, 0, 2]
    cls_loss = jnp.mean(cls_sum / jnp.maximum(npos, 1.0)).reshape(1)
    reg_loss = jnp.mean(reg_sum / jnp.maximum(npos * 4.0, 1.0)).reshape(1)
    return cls_loss, reg_loss


# speedup vs baseline: 4.4106x; 1.0100x over previous
import jax
import jax.numpy as jnp
from jax import lax
from jax.experimental import pallas as pl
from jax.experimental.pallas import tpu as pltpu

_BA = 10240


def _focal_body(cls_ref, reg_ref, anch_ref, ann_ref, nvalid_ref, out_ref):
    i = pl.program_id(1)

    x = jnp.clip(cls_ref[0], 1e-4, 1.0 - 1e-4)
    C, BA = x.shape
    nvalid = nvalid_ref[0]
    valid = (lax.broadcasted_iota(jnp.int32, (1, BA), 1) + i * BA) < nvalid
    x = jnp.where(valid, x, 0.5)
    annb = ann_ref[0]
    M = annb.shape[0]
    bx1 = annb[:, 0:1]
    by1 = annb[:, 1:2]
    bx2 = annb[:, 2:3]
    by2 = annb[:, 3:4]

    ax1 = jnp.where(valid, anch_ref[0:1, :], 0.0)
    ay1 = jnp.where(valid, anch_ref[1:2, :], 0.0)
    ax2 = jnp.where(valid, anch_ref[2:3, :], 16.0)
    ay2 = jnp.where(valid, anch_ref[3:4, :], 16.0)
    aw = ax2 - ax1
    ah = ay2 - ay1
    acx = ax1 + 0.5 * aw
    acy = ay1 + 0.5 * ah
    aw_s = jnp.maximum(aw, 1e-3)
    ah_s = jnp.maximum(ah, 1e-3)

    area_a = aw * ah
    area_b = (bx2 - bx1) * (by2 - by1)
    iw = jnp.maximum(jnp.minimum(ax2, bx2) - jnp.maximum(ax1, bx1), 0.0)
    ih = jnp.maximum(jnp.minimum(ay2, by2) - jnp.maximum(ay1, by1), 0.0)
    inter = iw * ih
    ua = jnp.maximum(area_a + area_b - inter, 1e-8)
    iou = inter / ua

    iou_max = jnp.max(iou, axis=0, keepdims=True)
    iota_m = lax.broadcasted_iota(jnp.int32, (M, BA), 0)
    amax = jnp.min(jnp.where(iou == iou_max, iota_m, M), axis=0, keepdims=True)
    oh_m = iota_m == amax

    def pick(col):
        return jnp.sum(jnp.where(oh_m, col, 0.0), axis=0, keepdims=True)

    gx1 = pick(bx1)
    gy1 = pick(by1)
    gx2 = pick(bx2)
    gy2 = pick(by2)

    pos = (iou_max >= 0.5) & valid
    contrib = ((iou_max >= 0.5) | (iou_max < 0.4)) & valid
    posf = pos.astype(jnp.float32)
    npos = jnp.sum(posf)

    neg = (0.75 * (x * x)) * (-jnp.log(1.0 - x))
    s_neg = jax.lax.dot_general(jnp.ones((1, C), jnp.float32), neg,
                                (((1,), (0,)), ((), ())),
                                preferred_element_type=jnp.float32)
    blab = annb[:, 4:5]
    glab = pick(blab)
    lab_i = glab.astype(jnp.int32)
    iota_c = lax.broadcasted_iota(jnp.int32, (C, BA), 0)
    x_sel = jnp.sum(jnp.where(iota_c == lab_i, x, 0.0), axis=0, keepdims=True)
    pos_term = (0.25 * (1.0 - x_sel) * (1.0 - x_sel)) * (-jnp.log(x_sel))
    neg_sel = (0.75 * (x_sel * x_sel)) * (-jnp.log(1.0 - x_sel))
    row_cls = (jnp.where(contrib, s_neg, 0.0)
               + jnp.where(pos, pos_term - neg_sel, 0.0))
    cls_s = jnp.sum(row_cls)

    gt_w = gx2 - gx1
    gt_h = gy2 - gy1
    gcx = gx1 + 0.5 * gt_w
    gcy = gy1 + 0.5 * gt_h
    gt_w = jnp.maximum(gt_w, 1.0)
    gt_h = jnp.maximum(gt_h, 1.0)
    tdx = ((gcx - acx) / aw_s) / 0.1
    tdy = ((gcy - acy) / ah_s) / 0.1
    tdw = jnp.log(gt_w / aw_s) / 0.2
    tdh = jnp.log(gt_h / ah_s) / 0.2

    def smooth_l1(t, c):
        d = jnp.abs(t - reg_ref[0, c:c + 1, :])
        return jnp.where(d <= 1.0 / 9.0, 0.5 * 9.0 * (d * d), d - 0.5 / 9.0)

    rl = smooth_l1(tdx, 0) + smooth_l1(tdy, 1) + smooth_l1(tdw, 2) + smooth_l1(tdh, 3)
    reg_s = jnp.sum(jnp.where(pos, rl, 0.0))

    @pl.when(i == 0)
    def _init():
        out_ref[0, 0, 0] = 0.0
        out_ref[0, 0, 1] = 0.0
        out_ref[0, 0, 2] = 0.0
        out_ref[0, 0, 3] = 0.0

    out_ref[0, 0, 0] += cls_s
    out_ref[0, 0, 1] += reg_s
    out_ref[0, 0, 2] += npos


@jax.jit
def kernel(classifications, regressions, anchors, annotations):
    B, A, C = classifications.shape
    M = annotations.shape[1]
    nblk = -(-A // _BA)

    cls_t = classifications.transpose(0, 2, 1)
    reg_t = regressions.transpose(0, 2, 1)
    anch_t = anchors[0].T
    nvalid = jnp.full((1,), A, dtype=jnp.int32)

    out = pl.pallas_call(
        _focal_body,
        grid=(B, nblk),
        in_specs=[
            pl.BlockSpec((1, C, _BA), lambda j, i: (j, 0, i)),
            pl.BlockSpec((1, 4, _BA), lambda j, i: (j, 0, i)),
            pl.BlockSpec((4, _BA), lambda j, i: (0, i)),
            pl.BlockSpec((1, M, 5), lambda j, i: (j, 0, 0)),
            pl.BlockSpec(memory_space=pltpu.SMEM),
        ],
        out_specs=pl.BlockSpec((1, 1, 4), lambda j, i: (j, 0, 0),
                               memory_space=pltpu.SMEM),
        out_shape=jax.ShapeDtypeStruct((B, 1, 4), jnp.float32),
    )(cls_t, reg_t, anch_t, annotations, nvalid)

    cls_sum = out[:, 0, 0]
    reg_sum = out[:, 0, 1]
    npos = out[:, 0, 2]
    cls_loss = jnp.mean(cls_sum / jnp.maximum(npos, 1.0)).reshape(1)
    reg_loss = jnp.mean(reg_sum / jnp.maximum(npos * 4.0, 1.0)).reshape(1)
    return cls_loss, reg_loss
